# Initial kernel scaffold; baseline (speedup 1.0000x reference)
import jax
import jax.numpy as jnp
from jax.experimental import pallas as pl


def kernel(x, edge_index, spatial_coords, ln1_w, ln1_b, a1_sW, a1_sb, a1_nW, a1_nb, a1_rW, a1_rb, beta1, ln2_w, ln2_b, a2_sW, a2_sb, a2_nW, a2_nb, a2_rW, a2_rb, beta2, red_W, red_b):
    raise NotImplementedError("write your pallas kernel here")


# speedup vs baseline: 1.7780x; 1.7780x over previous
import functools

import jax
import jax.numpy as jnp
from jax import lax
from jax.experimental import pallas as pl
from jax.experimental.pallas import tpu as pltpu
from jax.experimental.pallas import tpu_sc as plsc

_N = 10000
_K = 16
_D = 128
_Z = 64
_RADIUS = 1.0
_ALPHA = 2.0
_THR = 0.01

_NTILES = 32
_NPT = 320
_NPAD = _NTILES * _NPT
_NB = 8
_NBATCH = _NPT // _NB

_BR = 512


def _ln(x, w, b, eps=1e-5):
    mu = jnp.mean(x, axis=-1, keepdims=True)
    var = jnp.var(x, axis=-1, keepdims=True)
    return (x - mu) / jnp.sqrt(var + eps) * w + b


def _mm_t(a, w):
    return lax.dot_general(a, w, (((1,), (1,)), ((), ())),
                           preferred_element_type=jnp.float32)


def _leaky(x):
    return jnp.where(x >= 0, x, 0.01 * x)


def _tc_pre_body(x_ref, ln1w, ln1b, ln2w, ln2b, a1nW, a1nb, a1sW, a1sb,
                 a2nW, a2nb, b1, b2, T1_ref, T2_ref, tn1s_ref, s1_ref):
    x = x_ref[...]
    xn1 = _ln(x, ln1w[...], ln1b[...])
    xn2 = _ln(x, ln2w[...], ln2b[...])
    beta1 = b1[0, 0]
    beta2 = b2[0, 0]
    T1_ref[:, :_D] = _mm_t(xn1, a1nW[...]) + a1nb[...]
    T1_ref[:, _D:] = (1.0 - beta1) * xn1
    T2_ref[:, :_D] = _mm_t(xn2, a2nW[...]) + a2nb[...]
    T2_ref[:, _D:] = (1.0 - beta2) * xn2
    tn1s_ref[...] = beta1 * xn1
    s1_ref[...] = _mm_t(xn1, a1sW[...]) + a1sb[...]


def _tc_mid_body(ctx1_ref, a1rW, a1rb, ln2w, ln2b, a2sW, a2sb, b2,
                 s2_ref, tn2s_ref):
    r1 = _leaky(_leaky(_mm_t(ctx1_ref[...], a1rW[...]) + a1rb[...]))
    tn2 = _ln(r1, ln2w[...], ln2b[...])
    s2_ref[...] = _mm_t(tn2, a2sW[...]) + a2sb[...]
    tn2s_ref[...] = b2[0, 0] * tn2


def _tc_final_body(ctx2_ref, a2rW, a2rb, redW, redb, out_ref):
    r2 = _leaky(_leaky(_mm_t(ctx2_ref[...], a2rW[...]) + a2rb[...]))
    out_ref[...] = _leaky(_mm_t(r2, redW[...]) + redb[...])


def _row_spec(w):
    return pl.BlockSpec((_BR, w), lambda i: (i, 0))


def _full_spec(shape):
    return pl.BlockSpec(shape, lambda i: tuple(0 for _ in shape))


_GRID = (_NPAD // _BR,)

_tc_pre = pl.pallas_call(
    _tc_pre_body,
    grid=_GRID,
    in_specs=[_row_spec(_D)] + [_full_spec((1, _D))] * 4
             + [_full_spec((_D, _D)), _full_spec((1, _D))] * 3
             + [_full_spec((1, 1))] * 2,
    out_specs=[_row_spec(2 * _D), _row_spec(2 * _D), _row_spec(_D),
               _row_spec(_D)],
    out_shape=[
        jax.ShapeDtypeStruct((_NPAD, 2 * _D), jnp.float32),
        jax.ShapeDtypeStruct((_NPAD, 2 * _D), jnp.float32),
        jax.ShapeDtypeStruct((_NPAD, _D), jnp.float32),
        jax.ShapeDtypeStruct((_NPAD, _D), jnp.float32),
    ],
)

_tc_mid = pl.pallas_call(
    _tc_mid_body,
    grid=_GRID,
    in_specs=[_row_spec(_D), _full_spec((_D, _D)), _full_spec((1, _D)),
              _full_spec((1, _D)), _full_spec((1, _D)),
              _full_spec((_D, _D)), _full_spec((1, _D)),
              _full_spec((1, 1))],
    out_specs=[_row_spec(_D), _row_spec(_D)],
    out_shape=[
        jax.ShapeDtypeStruct((_NPAD, _D), jnp.float32),
        jax.ShapeDtypeStruct((_NPAD, _D), jnp.float32),
    ],
)

_tc_final = pl.pallas_call(
    _tc_final_body,
    grid=_GRID,
    in_specs=[_row_spec(_D), _full_spec((_D, _D)), _full_spec((1, _D)),
              _full_spec((_Z, _D)), _full_spec((1, _Z))],
    out_specs=_row_spec(_Z),
    out_shape=jax.ShapeDtypeStruct((_NPAD, _Z), jnp.float32),
)


def _c(v, dtype=jnp.float32):
    return jnp.full((16,), v, dtype)


def _make_sc_block():
    mesh = plsc.VectorSubcoreMesh(core_axis_name="c", subcore_axis_name="s",
                                  num_cores=2, num_subcores=16)

    @functools.partial(
        pl.kernel,
        mesh=mesh,
        compiler_params=pltpu.CompilerParams(needs_layout_passes=False),
        out_type=jax.ShapeDtypeStruct((_NPAD * _D,), jnp.float32),
        scratch_types=[
            pltpu.VMEM((128,), jnp.int32),
            pltpu.VMEM((128, 2 * _D), jnp.float32),
            pltpu.VMEM((_NB * _D,), jnp.float32),
            pltpu.VMEM((_NB * _D,), jnp.float32),
            pltpu.VMEM((_NB * _D,), jnp.float32),
            pltpu.VMEM((_NPAD * 2 + 16,), jnp.float32),
            pltpu.SemaphoreType.DMA,
        ],
    )
    def sc_block(T_hbm, s_hbm, tn_hbm, col_hbm, coords_hbm, ctx_hbm,
                 idx_v, rows_v, s_v, tn_v, out_v, coords_v, sem):
        wid = lax.axis_index("s") * 2 + lax.axis_index("c")
        base = wid * _NPT
        pltpu.sync_copy(coords_hbm, coords_v.at[pl.ds(0, _NPAD * 2)])

        neg_scale = _c(-_ALPHA / (_RADIUS + 1e-8))
        thr = _c(_THR)
        zero = _c(0.0)
        one = _c(1.0)
        c_half = _c(0.5)
        c_3h = _c(1.5)
        magic = _c(0x5F3759DF, jnp.int32)
        one_i = _c(1, jnp.int32)
        two_i = _c(2, jnp.int32)

        def batch_body(b, carry):
            nb = base + b * _NB
            pltpu.sync_copy(col_hbm.at[pl.ds(nb * _K, _NB * _K)], idx_v)
            pltpu.async_copy(T_hbm.at[idx_v], rows_v, sem).wait()
            pltpu.sync_copy(s_hbm.at[pl.ds(nb * _D, _NB * _D)], s_v)
            pltpu.sync_copy(tn_hbm.at[pl.ds(nb * _D, _NB * _D)], tn_v)

            def node_body(j, c2):
                node = nb + j
                colv = idx_v[pl.ds(j * _K, 16)]
                ci = colv * two_i
                cgx = plsc.load_gather(coords_v, [ci])
                cgy = plsc.load_gather(coords_v, [ci + one_i])
                cxy = coords_v[pl.ds(node * 2, 16)]
                cx = jnp.full((16,), cxy[0], jnp.float32)
                cy = jnp.full((16,), cxy[1], jnp.float32)
                dx = cx - cgx
                dy = cy - cgy
                z = dx * dx + dy * dy
                y = plsc.bitcast(
                    magic - lax.shift_right_logical(plsc.bitcast(z, jnp.int32),
                                                    one_i),
                    jnp.float32)
                hz = c_half * z
                y = y * (c_3h - hz * y * y)
                y = y * (c_3h - hz * y * y)
                y = y * (c_3h - hz * y * y)
                dwv = jnp.exp(z * y * neg_scale)
                dwb = [jnp.full((16,), dwv[k], jnp.float32)
                       for k in range(_K)]
                for cc in range(_D // 16):
                    off = j * _D + cc * 16
                    es = jnp.exp(s_v[pl.ds(off, 16)])
                    zs = es
                    elist = []
                    for k in range(_K):
                        gk = rows_v[j * _K + k, pl.ds(cc * 16, 16)] * dwb[k]
                        ek = jnp.exp(gk)
                        elist.append(ek)
                        zs = zs + ek
                    recip = one / zs
                    swc = es * recip
                    swc = jnp.where(swc >= thr, swc, zero)
                    acc = swc * tn_v[pl.ds(off, 16)]
                    for k in range(_K):
                        wk = elist[k] * recip
                        wk = jnp.where(wk >= thr, wk, zero)
                        acc = acc + wk * rows_v[j * _K + k,
                                                pl.ds(_D + cc * 16, 16)]
                    out_v[pl.ds(off, 16)] = acc
                return c2

            lax.fori_loop(0, _NB, node_body, 0)
            pltpu.sync_copy(out_v, ctx_hbm.at[pl.ds(nb * _D, _NB * _D)])
            return carry

        lax.fori_loop(0, _NBATCH, batch_body, 0)

    return sc_block


_sc_block = _make_sc_block()


@jax.jit
def kernel(x, edge_index, spatial_coords, ln1_w, ln1_b, a1_sW, a1_sb, a1_nW,
           a1_nb, a1_rW, a1_rb, beta1, ln2_w, ln2_b, a2_sW, a2_sb, a2_nW,
           a2_nb, a2_rW, a2_rb, beta2, red_W, red_b):
    pad_n = _NPAD - _N
    xp = jnp.pad(x, ((0, pad_n), (0, 0)))
    col = jnp.pad(edge_index[1], (0, pad_n * _K)).astype(jnp.int32)
    coords = jnp.pad(spatial_coords, ((0, pad_n), (0, 0))).reshape(-1)

    r2 = lambda a: a.reshape(1, -1)
    b1 = jnp.asarray(beta1, jnp.float32).reshape(1, 1)
    b2 = jnp.asarray(beta2, jnp.float32).reshape(1, 1)

    T1, T2, tn1s, s1 = _tc_pre(
        xp, r2(ln1_w), r2(ln1_b), r2(ln2_w), r2(ln2_b),
        a1_nW, r2(a1_nb), a1_sW, r2(a1_sb), a2_nW, r2(a2_nb), b1, b2)

    ctx1 = _sc_block(T1, s1.reshape(-1), tn1s.reshape(-1), col, coords)
    ctx1 = ctx1.reshape(_NPAD, _D)

    s2, tn2s = _tc_mid(ctx1, a1_rW, r2(a1_rb), r2(ln2_w), r2(ln2_b),
                       a2_sW, r2(a2_sb), b2)

    ctx2 = _sc_block(T2, s2.reshape(-1), tn2s.reshape(-1), col, coords)
    ctx2 = ctx2.reshape(_NPAD, _D)

    out = _tc_final(ctx2, a2_rW, r2(a2_rb), red_W, r2(red_b))
    return out[:_N]


# speedup vs baseline: 2.6062x; 1.4658x over previous
import functools

import jax
import jax.numpy as jnp
from jax import lax
from jax.experimental import pallas as pl
from jax.experimental.pallas import tpu as pltpu
from jax.experimental.pallas import tpu_sc as plsc

_N = 10000
_K = 16
_D = 128
_Z = 64
_RADIUS = 1.0
_ALPHA = 2.0
_THR = 0.01

_NTILES = 32
_NPT = 320
_NPAD = _NTILES * _NPT
_NB = 8
_NBATCH = _NPT // _NB

_BR = 512


def _ln(x, w, b, eps=1e-5):
    mu = jnp.mean(x, axis=-1, keepdims=True)
    var = jnp.var(x, axis=-1, keepdims=True)
    return (x - mu) / jnp.sqrt(var + eps) * w + b


def _mm_t(a, w):
    return lax.dot_general(a, w, (((1,), (1,)), ((), ())),
                           preferred_element_type=jnp.float32)


def _leaky(x):
    return jnp.where(x >= 0, x, 0.01 * x)


def _tc_pre_body(x_ref, ln1w, ln1b, ln2w, ln2b, a1nW, a1nb, a1sW, a1sb,
                 a2nW, a2nb, b1, b2, T1_ref, T2_ref, st1_ref):
    x = x_ref[...]
    xn1 = _ln(x, ln1w[...], ln1b[...])
    xn2 = _ln(x, ln2w[...], ln2b[...])
    beta1 = b1[0, 0]
    beta2 = b2[0, 0]
    T1_ref[:, :_D] = _mm_t(xn1, a1nW[...]) + a1nb[...]
    T1_ref[:, _D:] = (1.0 - beta1) * xn1
    T2_ref[:, :_D] = _mm_t(xn2, a2nW[...]) + a2nb[...]
    T2_ref[:, _D:] = (1.0 - beta2) * xn2
    st1_ref[:, :_D] = _mm_t(xn1, a1sW[...]) + a1sb[...]
    st1_ref[:, _D:] = beta1 * xn1


def _tc_mid_body(ctx1_ref, a1rW, a1rb, ln2w, ln2b, a2sW, a2sb, b2, st2_ref):
    r1 = _leaky(_leaky(_mm_t(ctx1_ref[...], a1rW[...]) + a1rb[...]))
    tn2 = _ln(r1, ln2w[...], ln2b[...])
    st2_ref[:, :_D] = _mm_t(tn2, a2sW[...]) + a2sb[...]
    st2_ref[:, _D:] = b2[0, 0] * tn2


def _tc_final_body(ctx2_ref, a2rW, a2rb, redW, redb, out_ref):
    r2 = _leaky(_leaky(_mm_t(ctx2_ref[...], a2rW[...]) + a2rb[...]))
    out_ref[...] = _leaky(_mm_t(r2, redW[...]) + redb[...])


def _row_spec(w):
    return pl.BlockSpec((_BR, w), lambda i: (i, 0))


def _full_spec(shape):
    return pl.BlockSpec(shape, lambda i: tuple(0 for _ in shape))


_GRID = (_NPAD // _BR,)

_tc_pre = pl.pallas_call(
    _tc_pre_body,
    grid=_GRID,
    in_specs=[_row_spec(_D)] + [_full_spec((1, _D))] * 4
             + [_full_spec((_D, _D)), _full_spec((1, _D))] * 3
             + [_full_spec((1, 1))] * 2,
    out_specs=[_row_spec(2 * _D), _row_spec(2 * _D), _row_spec(2 * _D)],
    out_shape=[
        jax.ShapeDtypeStruct((_NPAD, 2 * _D), jnp.float32),
        jax.ShapeDtypeStruct((_NPAD, 2 * _D), jnp.float32),
        jax.ShapeDtypeStruct((_NPAD, 2 * _D), jnp.float32),
    ],
)

_tc_mid = pl.pallas_call(
    _tc_mid_body,
    grid=_GRID,
    in_specs=[_row_spec(_D), _full_spec((_D, _D)), _full_spec((1, _D)),
              _full_spec((1, _D)), _full_spec((1, _D)),
              _full_spec((_D, _D)), _full_spec((1, _D)),
              _full_spec((1, 1))],
    out_specs=_row_spec(2 * _D),
    out_shape=jax.ShapeDtypeStruct((_NPAD, 2 * _D), jnp.float32),
)

_tc_final = pl.pallas_call(
    _tc_final_body,
    grid=_GRID,
    in_specs=[_row_spec(_D), _full_spec((_D, _D)), _full_spec((1, _D)),
              _full_spec((_Z, _D)), _full_spec((1, _Z))],
    out_specs=_row_spec(_Z),
    out_shape=jax.ShapeDtypeStruct((_NPAD, _Z), jnp.float32),
)


def _c(v, dtype=jnp.float32):
    return jnp.full((16,), v, dtype)


def _make_sc_block():
    mesh = plsc.VectorSubcoreMesh(core_axis_name="c", subcore_axis_name="s",
                                  num_cores=2, num_subcores=16)

    @functools.partial(
        pl.kernel,
        mesh=mesh,
        compiler_params=pltpu.CompilerParams(needs_layout_passes=False),
        out_type=jax.ShapeDtypeStruct((_NPAD * _D,), jnp.float32),
        scratch_types=[
            pltpu.VMEM((128,), jnp.int32),
            pltpu.VMEM((128,), jnp.int32),
            pltpu.VMEM((128, 2 * _D), jnp.float32),
            pltpu.VMEM((128, 2 * _D), jnp.float32),
            pltpu.VMEM((_NB * 2 * _D,), jnp.float32),
            pltpu.VMEM((_NB * 2 * _D,), jnp.float32),
            pltpu.VMEM((_NB * _D,), jnp.float32),
            pltpu.VMEM((_NPAD * 2 + 16,), jnp.float32),
            pltpu.SemaphoreType.DMA,
            pltpu.SemaphoreType.DMA,
            pltpu.SemaphoreType.DMA,
            pltpu.SemaphoreType.DMA,
        ],
    )
    def sc_block(T_hbm, st_hbm, col_hbm, coords_hbm, ctx_hbm,
                 idx_a, idx_b, rows_a, rows_b, st_a, st_b, out_v, coords_v,
                 sem_a, sem_b, sem_sa, sem_sb):
        wid = lax.axis_index("s") * 2 + lax.axis_index("c")
        base = wid * _NPT
        pltpu.sync_copy(coords_hbm, coords_v.at[pl.ds(0, _NPAD * 2)])

        neg_scale = _c(-_ALPHA / (_RADIUS + 1e-8))
        thr = _c(_THR)
        zero = _c(0.0)
        one = _c(1.0)
        c_half = _c(0.5)
        c_3h = _c(1.5)
        magic = _c(0x5F3759DF, jnp.int32)
        one_i = _c(1, jnp.int32)
        two_i = _c(2, jnp.int32)

        def issue(b, idx_v, rows_v, st_v, sem, sem_s):
            nb = base + b * _NB
            pltpu.sync_copy(col_hbm.at[pl.ds(nb * _K, _NB * _K)], idx_v)
            pltpu.async_copy(T_hbm.at[idx_v], rows_v, sem)
            pltpu.async_copy(st_hbm.at[pl.ds(nb * 2 * _D, _NB * 2 * _D)],
                             st_v, sem_s)

        def process(b, idx_v, rows_v, st_v, sem, sem_s):
            nb = base + b * _NB
            pltpu.make_async_copy(
                st_hbm.at[pl.ds(nb * 2 * _D, _NB * 2 * _D)], st_v,
                sem_s).wait()
            pltpu.make_async_copy(T_hbm.at[idx_v], rows_v, sem).wait()

            def node_body(j, c2):
                node = nb + j
                colv = idx_v[pl.ds(j * _K, 16)]
                ci = colv * two_i
                cgx = plsc.load_gather(coords_v, [ci])
                cgy = plsc.load_gather(coords_v, [ci + one_i])
                cxy = coords_v[pl.ds(node * 2, 16)]
                cx = jnp.full((16,), cxy[0], jnp.float32)
                cy = jnp.full((16,), cxy[1], jnp.float32)
                dx = cx - cgx
                dy = cy - cgy
                z = dx * dx + dy * dy
                y = plsc.bitcast(
                    magic - lax.shift_right_logical(plsc.bitcast(z, jnp.int32),
                                                    one_i),
                    jnp.float32)
                hz = c_half * z
                y = y * (c_3h - hz * y * y)
                y = y * (c_3h - hz * y * y)
                y = y * (c_3h - hz * y * y)
                dwv = jnp.exp(z * y * neg_scale)
                dwb = [jnp.full((16,), dwv[k], jnp.float32)
                       for k in range(_K)]
                for cc in range(_D // 16):
                    off = j * _D + cc * 16
                    soff = j * 2 * _D + cc * 16
                    es = jnp.exp(st_v[pl.ds(soff, 16)])
                    zs = es
                    elist = []
                    for k in range(_K):
                        gk = rows_v[j * _K + k, pl.ds(cc * 16, 16)] * dwb[k]
                        ek = jnp.exp(gk)
                        elist.append(ek)
                        zs = zs + ek
                    recip = one / zs
                    tz = thr * zs
                    acc = (jnp.where(es >= tz, es, zero)
                           * st_v[pl.ds(soff + _D, 16)])
                    for k in range(_K):
                        wk = jnp.where(elist[k] >= tz, elist[k], zero)
                        acc = acc + wk * rows_v[j * _K + k,
                                                pl.ds(_D + cc * 16, 16)]
                    out_v[pl.ds(off, 16)] = acc * recip
                return c2

            lax.fori_loop(0, _NB, node_body, 0)
            pltpu.sync_copy(out_v, ctx_hbm.at[pl.ds(nb * _D, _NB * _D)])

        issue(0, idx_a, rows_a, st_a, sem_a, sem_sa)

        def pair_body(i, carry):
            b0 = i * 2
            issue(b0 + 1, idx_b, rows_b, st_b, sem_b, sem_sb)
            process(b0, idx_a, rows_a, st_a, sem_a, sem_sa)

            @pl.when(b0 + 2 < _NBATCH)
            def _():
                issue(b0 + 2, idx_a, rows_a, st_a, sem_a, sem_sa)

            process(b0 + 1, idx_b, rows_b, st_b, sem_b, sem_sb)
            return carry

        lax.fori_loop(0, _NBATCH // 2, pair_body, 0)

    return sc_block


_sc_block = _make_sc_block()


@jax.jit
def kernel(x, edge_index, spatial_coords, ln1_w, ln1_b, a1_sW, a1_sb, a1_nW,
           a1_nb, a1_rW, a1_rb, beta1, ln2_w, ln2_b, a2_sW, a2_sb, a2_nW,
           a2_nb, a2_rW, a2_rb, beta2, red_W, red_b):
    pad_n = _NPAD - _N
    xp = jnp.pad(x, ((0, pad_n), (0, 0)))
    col = jnp.pad(edge_index[1], (0, pad_n * _K)).astype(jnp.int32)
    coords = jnp.pad(spatial_coords, ((0, pad_n), (0, 0))).reshape(-1)

    r2 = lambda a: a.reshape(1, -1)
    b1 = jnp.asarray(beta1, jnp.float32).reshape(1, 1)
    b2 = jnp.asarray(beta2, jnp.float32).reshape(1, 1)

    T1, T2, st1 = _tc_pre(
        xp, r2(ln1_w), r2(ln1_b), r2(ln2_w), r2(ln2_b),
        a1_nW, r2(a1_nb), a1_sW, r2(a1_sb), a2_nW, r2(a2_nb), b1, b2)

    ctx1 = _sc_block(T1, st1.reshape(-1), col, coords)
    ctx1 = ctx1.reshape(_NPAD, _D)

    st2 = _tc_mid(ctx1, a1_rW, r2(a1_rb), r2(ln2_w), r2(ln2_b),
                  a2_sW, r2(a2_sb), b2)

    ctx2 = _sc_block(T2, st2.reshape(-1), col, coords)
    ctx2 = ctx2.reshape(_NPAD, _D)

    out = _tc_final(ctx2, a2_rW, r2(a2_rb), red_W, r2(red_b))
    return out[:_N]
